# baseline (speedup 1.0000x reference)
import functools

import jax
import jax.numpy as jnp
from jax import lax
from jax.experimental import pallas as pl
from jax.experimental.pallas import tpu as pltpu
from jax.experimental.pallas import tpu_sc as plsc

N = 10000
E = 320000
D = 128
H = 128
C = 10
G = 64

NC = 2
NS = 16
NW = NC * NS
E_PER_W = E // NW
CHUNK = 50
N_CHUNKS = E_PER_W // CHUNK
NBUF = 4
RC = 8
ROUNDS = N_CHUNKS // RC
ROW_STEP = 624
ROW_COPY = 640


def _agg_body(h_hbm, src_hbm, dst_hbm, zero_hbm, out_hbm,
              src_v, dst_v, rows_v, acc, g0, g1, g2, g3, s0, s1, s2, s3):
    gsems = (g0, g1, g2, g3)
    ssems = (s0, s1, s2, s3)
    cid = lax.axis_index("c")
    sid = lax.axis_index("s")
    wid = sid * NC + cid

    r0 = sid * ROW_STEP

    @pl.when(cid == 0)
    def _():
        pltpu.sync_copy(h_hbm.at[pl.ds(r0, ROW_COPY)],
                        acc.at[pl.ds(r0, ROW_COPY)])

    @pl.when(cid != 0)
    def _():
        pltpu.sync_copy(zero_hbm.at[pl.ds(r0, ROW_COPY)],
                        acc.at[pl.ds(r0, ROW_COPY)])

    plsc.subcore_barrier()

    def round_(r, carry):
        base = r * RC
        pltpu.sync_copy(src_hbm.at[wid, pl.ds(base, RC)], src_v)
        pltpu.sync_copy(dst_hbm.at[wid, pl.ds(base, RC)], dst_v)
        g = {b: pltpu.async_copy(h_hbm.at[src_v.at[b]], rows_v.at[b],
                                 gsems[b]) for b in range(NBUF)}
        s = {}
        for j in range(RC):
            b = j % NBUF
            g[b].wait()
            s[b] = pltpu.async_copy(rows_v.at[b], acc.at[dst_v.at[j]],
                                    ssems[b], add=True)
            nj = j + NBUF
            if nj < RC:
                s[b].wait()
                g[b] = pltpu.async_copy(h_hbm.at[src_v.at[nj]],
                                        rows_v.at[b], gsems[b])
        for j in range(RC - NBUF, RC):
            s[j % NBUF].wait()
        return carry

    lax.fori_loop(0, ROUNDS, round_, 0)

    plsc.subcore_barrier()

    pltpu.sync_copy(acc.at[pl.ds(r0, ROW_COPY)],
                    out_hbm.at[pl.ds(cid * N + r0, ROW_COPY)])


_agg = pl.kernel(
    _agg_body,
    out_type=jax.ShapeDtypeStruct((2 * N, D), jnp.float32),
    mesh=plsc.VectorSubcoreMesh(core_axis_name="c", subcore_axis_name="s"),
    scratch_types=[
        pltpu.VMEM((RC, CHUNK), jnp.int32),
        pltpu.VMEM((RC, CHUNK), jnp.int32),
        pltpu.VMEM((NBUF, CHUNK, D), jnp.float32),
        pltpu.VMEM_SHARED((N, D), jnp.float32),
        pltpu.SemaphoreType.DMA,
        pltpu.SemaphoreType.DMA,
        pltpu.SemaphoreType.DMA,
        pltpu.SemaphoreType.DMA,
        pltpu.SemaphoreType.DMA,
        pltpu.SemaphoreType.DMA,
        pltpu.SemaphoreType.DMA,
        pltpu.SemaphoreType.DMA,
    ],
)


def _layer_body(agg_ref, batch_ref, W1_ref, b1_ref, g_ref, be_ref,
                W2_ref, b2_ref, out_ref, p_ref):
    z = agg_ref[:N, :] + agg_ref[N:, :]
    y = jnp.dot(z, W1_ref[...], preferred_element_type=jnp.float32) + b1_ref[...]
    m = jnp.mean(y, axis=0, keepdims=True)
    yc = y - m
    v = jnp.mean(yc * yc, axis=0, keepdims=True)
    zn = yc * lax.rsqrt(v + 1e-5) * g_ref[...] + be_ref[...]
    zr = jnp.maximum(zn, 0.0)
    h2 = jnp.dot(zr, W2_ref[...], preferred_element_type=jnp.float32) + b2_ref[...]
    h2 = jnp.maximum(h2, 0.0)
    out_ref[...] = h2
    onehot = (batch_ref[...] == lax.broadcasted_iota(jnp.int32, (N, G), 1))
    p_ref[...] = lax.dot_general(onehot.astype(jnp.float32), h2,
                                 (((0,), (0,)), ((), ())),
                                 preferred_element_type=jnp.float32)


_layer = pl.pallas_call(
    _layer_body,
    out_shape=[jax.ShapeDtypeStruct((N, H), jnp.float32),
               jax.ShapeDtypeStruct((G, H), jnp.float32)],
)


def _readout_body(p1_ref, p2_ref, p3_ref, l1W_ref, l1b_ref, l2W_ref, l2b_ref,
                  out_ref):
    hcat = jnp.concatenate([p1_ref[...], p2_ref[...], p3_ref[...]], axis=1)
    hh = jnp.dot(hcat, l1W_ref[...], preferred_element_type=jnp.float32)
    hh = jnp.maximum(hh + l1b_ref[...], 0.0)
    out_ref[...] = (jnp.dot(hh, l2W_ref[...], preferred_element_type=jnp.float32)
                    + l2b_ref[...])


_readout = pl.pallas_call(
    _readout_body,
    out_shape=jax.ShapeDtypeStruct((G, C), jnp.float32),
)


def kernel(x, edge_index, batch,
           c1_W1, c1_b1, c1_g, c1_be, c1_W2, c1_b2,
           c2_W1, c2_b1, c2_g, c2_be, c2_W2, c2_b2,
           c3_W1, c3_b1, c3_g, c3_be, c3_W2, c3_b2,
           l1_W, l1_b, l2_W, l2_b):
    src = edge_index[0].reshape(NW, N_CHUNKS, CHUNK)
    dst = edge_index[1].reshape(NW, N_CHUNKS, CHUNK)
    zeros = jnp.zeros((N, D), jnp.float32)
    batch2 = batch.reshape(N, 1)

    def layer(h, W1, b1, g, be, W2, b2):
        agg2 = _agg(h, src, dst, zeros)
        return _layer(agg2, batch2, W1, b1.reshape(1, H), g.reshape(1, H),
                      be.reshape(1, H), W2, b2.reshape(1, H))

    h1, p1 = layer(x, c1_W1, c1_b1, c1_g, c1_be, c1_W2, c1_b2)
    h2, p2 = layer(h1, c2_W1, c2_b1, c2_g, c2_be, c2_W2, c2_b2)
    h3, p3 = layer(h2, c3_W1, c3_b1, c3_g, c3_be, c3_W2, c3_b2)

    return _readout(p1, p2, p3, l1_W, l1_b.reshape(1, 3 * H),
                    l2_W, l2_b.reshape(1, C))


# speedup vs baseline: 1.2416x; 1.2416x over previous
import functools

import jax
import jax.numpy as jnp
from jax import lax
from jax.experimental import pallas as pl
from jax.experimental.pallas import tpu as pltpu
from jax.experimental.pallas import tpu_sc as plsc

N = 10000
E = 320000
D = 128
H = 128
C = 10
G = 64

NC = 2
NS = 16
NW = NC * NS
E_PER_W = E // NW
CHUNK = 100
N_CHUNKS = E_PER_W // CHUNK
NBUF = 3
RC = 10
ROUNDS = N_CHUNKS // RC
ROW_STEP = 624
ROW_COPY = 640


def _agg_body(h_hbm, src_hbm, dst_hbm, zero_hbm, out_hbm,
              src_v, dst_v, rows_v, acc, g0, g1, g2, s0, s1, s2):
    gsems = (g0, g1, g2)
    ssems = (s0, s1, s2)
    cid = lax.axis_index("c")
    sid = lax.axis_index("s")
    wid = sid * NC + cid

    r0 = sid * ROW_STEP

    @pl.when(cid == 0)
    def _():
        pltpu.sync_copy(h_hbm.at[pl.ds(r0, ROW_COPY)],
                        acc.at[pl.ds(r0, ROW_COPY)])

    @pl.when(cid != 0)
    def _():
        pltpu.sync_copy(zero_hbm.at[pl.ds(r0, ROW_COPY)],
                        acc.at[pl.ds(r0, ROW_COPY)])

    plsc.subcore_barrier()

    def round_(r, carry):
        pltpu.sync_copy(src_hbm.at[wid * ROUNDS + r], src_v)
        pltpu.sync_copy(dst_hbm.at[wid * ROUNDS + r], dst_v)
        g = {b: pltpu.async_copy(h_hbm.at[src_v.at[b]], rows_v.at[b],
                                 gsems[b]) for b in range(NBUF)}
        s = {}
        for j in range(RC):
            b = j % NBUF
            g[b].wait()
            s[b] = pltpu.async_copy(rows_v.at[b], acc.at[dst_v.at[j]],
                                    ssems[b], add=True)
            nj = j + NBUF
            if nj < RC:
                s[b].wait()
                g[b] = pltpu.async_copy(h_hbm.at[src_v.at[nj]],
                                        rows_v.at[b], gsems[b])
        for j in range(RC - NBUF, RC):
            s[j % NBUF].wait()
        return carry

    lax.fori_loop(0, ROUNDS, round_, 0)

    plsc.subcore_barrier()

    pltpu.sync_copy(acc.at[pl.ds(r0, ROW_COPY)],
                    out_hbm.at[pl.ds(cid * N + r0, ROW_COPY)])


_agg = pl.kernel(
    _agg_body,
    out_type=jax.ShapeDtypeStruct((2 * N, D), jnp.float32),
    mesh=plsc.VectorSubcoreMesh(core_axis_name="c", subcore_axis_name="s"),
    scratch_types=[
        pltpu.VMEM((RC, CHUNK), jnp.int32),
        pltpu.VMEM((RC, CHUNK), jnp.int32),
        pltpu.VMEM((NBUF, CHUNK, D), jnp.float32),
        pltpu.VMEM_SHARED((N, D), jnp.float32),
        pltpu.SemaphoreType.DMA,
        pltpu.SemaphoreType.DMA,
        pltpu.SemaphoreType.DMA,
        pltpu.SemaphoreType.DMA,
        pltpu.SemaphoreType.DMA,
        pltpu.SemaphoreType.DMA,
    ],
)


def _layer_body(agg_ref, batch_ref, W1_ref, b1_ref, g_ref, be_ref,
                W2_ref, b2_ref, out_ref, p_ref):
    z = agg_ref[:N, :] + agg_ref[N:, :]
    y = jnp.dot(z, W1_ref[...], preferred_element_type=jnp.float32) + b1_ref[...]
    m = jnp.mean(y, axis=0, keepdims=True)
    yc = y - m
    v = jnp.mean(yc * yc, axis=0, keepdims=True)
    zn = yc * lax.rsqrt(v + 1e-5) * g_ref[...] + be_ref[...]
    zr = jnp.maximum(zn, 0.0)
    h2 = jnp.dot(zr, W2_ref[...], preferred_element_type=jnp.float32) + b2_ref[...]
    h2 = jnp.maximum(h2, 0.0)
    out_ref[...] = h2
    onehot = (batch_ref[...] == lax.broadcasted_iota(jnp.int32, (N, G), 1))
    p_ref[...] = lax.dot_general(onehot.astype(jnp.float32), h2,
                                 (((0,), (0,)), ((), ())),
                                 preferred_element_type=jnp.float32)


_layer = pl.pallas_call(
    _layer_body,
    out_shape=[jax.ShapeDtypeStruct((N, H), jnp.float32),
               jax.ShapeDtypeStruct((G, H), jnp.float32)],
)


def _readout_body(p1_ref, p2_ref, p3_ref, l1W_ref, l1b_ref, l2W_ref, l2b_ref,
                  out_ref):
    hcat = jnp.concatenate([p1_ref[...], p2_ref[...], p3_ref[...]], axis=1)
    hh = jnp.dot(hcat, l1W_ref[...], preferred_element_type=jnp.float32)
    hh = jnp.maximum(hh + l1b_ref[...], 0.0)
    out_ref[...] = (jnp.dot(hh, l2W_ref[...], preferred_element_type=jnp.float32)
                    + l2b_ref[...])


_readout = pl.pallas_call(
    _readout_body,
    out_shape=jax.ShapeDtypeStruct((G, C), jnp.float32),
)


def kernel(x, edge_index, batch,
           c1_W1, c1_b1, c1_g, c1_be, c1_W2, c1_b2,
           c2_W1, c2_b1, c2_g, c2_be, c2_W2, c2_b2,
           c3_W1, c3_b1, c3_g, c3_be, c3_W2, c3_b2,
           l1_W, l1_b, l2_W, l2_b):
    src = edge_index[0].reshape(NW * ROUNDS, RC, CHUNK)
    dst = edge_index[1].reshape(NW * ROUNDS, RC, CHUNK)
    zeros = jnp.zeros((N, D), jnp.float32)
    batch2 = batch.reshape(N, 1)

    def layer(h, W1, b1, g, be, W2, b2):
        agg2 = _agg(h, src, dst, zeros)
        return _layer(agg2, batch2, W1, b1.reshape(1, H), g.reshape(1, H),
                      be.reshape(1, H), W2, b2.reshape(1, H))

    h1, p1 = layer(x, c1_W1, c1_b1, c1_g, c1_be, c1_W2, c1_b2)
    h2, p2 = layer(h1, c2_W1, c2_b1, c2_g, c2_be, c2_W2, c2_b2)
    h3, p3 = layer(h2, c3_W1, c3_b1, c3_g, c3_be, c3_W2, c3_b2)

    return _readout(p1, p2, p3, l1_W, l1_b.reshape(1, 3 * H),
                    l2_W, l2_b.reshape(1, C))
